# baseline (device time: 36275 ns/iter reference)
import jax
import jax.numpy as jnp
from jax import lax
from jax.experimental import pallas as pl
from jax.experimental.pallas import tpu as pltpu

N_DEV = 32
ZG = 4
PB = 8


def kernel(x, w_mat):
    m_per, k = x.shape
    _, n = w_mat.shape
    n_per = n // N_DEV

    def body(x_ref, w_ref, out_ref, staged, rbuf, send1, recv1, send2, recv2):
        my = lax.axis_index("i")
        g = lax.div(my, PB)
        b = lax.rem(my, PB)

        barrier_sem = pltpu.get_barrier_semaphore()
        for db in range(1, PB):
            pl.semaphore_signal(
                barrier_sem, inc=1,
                device_id=(g * PB + lax.rem(b + db, PB),),
                device_id_type=pl.DeviceIdType.MESH,
            )
        for dz in range(1, ZG):
            pl.semaphore_signal(
                barrier_sem, inc=1,
                device_id=(lax.rem(g + dz, ZG) * PB + b,),
                device_id_type=pl.DeviceIdType.MESH,
            )

        y = jnp.maximum(
            jnp.dot(x_ref[...], w_ref[...], preferred_element_type=jnp.float32),
            0.0,
        )
        for b2 in range(PB):
            for zt in range(ZG):
                staged[b2, zt] = y[:, (zt * PB + b2) * n_per:
                                   (zt * PB + b2 + 1) * n_per]

        rbuf[:, pl.ds(b * m_per, m_per), :] = staged[b]

        pl.semaphore_wait(barrier_sem, PB - 1 + ZG - 1)

        p1 = []
        for db in range(1, PB):
            b2 = lax.rem(b + db, PB)
            rdma = pltpu.make_async_remote_copy(
                src_ref=staged.at[b2],
                dst_ref=rbuf.at[:, pl.ds(b * m_per, m_per), :],
                send_sem=send1.at[db],
                recv_sem=recv1.at[b],
                device_id=(g * PB + b2,),
                device_id_type=pl.DeviceIdType.MESH,
            )
            rdma.start()
            p1.append(rdma)
        for db in range(1, PB):
            bs = lax.rem(b - db + PB, PB)
            pltpu.make_async_remote_copy(
                src_ref=staged.at[bs],
                dst_ref=rbuf.at[:, pl.ds(bs * m_per, m_per), :],
                send_sem=send1.at[db],
                recv_sem=recv1.at[bs],
                device_id=(g * PB + bs,),
                device_id_type=pl.DeviceIdType.MESH,
            ).wait_recv()

        out_ref[pl.ds(g * PB * m_per, PB * m_per), :] = rbuf[g]

        p2 = []
        for dz in range(1, ZG):
            g2 = lax.rem(g + dz, ZG)
            rdma = pltpu.make_async_remote_copy(
                src_ref=rbuf.at[g2],
                dst_ref=out_ref.at[pl.ds(g * PB * m_per, PB * m_per), :],
                send_sem=send2.at[dz],
                recv_sem=recv2.at[g],
                device_id=(g2 * PB + b,),
                device_id_type=pl.DeviceIdType.MESH,
            )
            rdma.start()
            p2.append(rdma)
        for dz in range(1, ZG):
            gs = lax.rem(g - dz + ZG, ZG)
            pltpu.make_async_remote_copy(
                src_ref=rbuf.at[gs],
                dst_ref=out_ref.at[pl.ds(gs * PB * m_per, PB * m_per), :],
                send_sem=send2.at[dz],
                recv_sem=recv2.at[gs],
                device_id=(gs * PB + b,),
                device_id_type=pl.DeviceIdType.MESH,
            ).wait_recv()

        for rdma in p1:
            rdma.wait_send()
        for rdma in p2:
            rdma.wait_send()

    return pl.pallas_call(
        body,
        out_shape=jax.ShapeDtypeStruct((N_DEV * m_per, n_per), jnp.float32),
        in_specs=[
            pl.BlockSpec(memory_space=pltpu.VMEM),
            pl.BlockSpec(memory_space=pltpu.VMEM),
        ],
        out_specs=pl.BlockSpec(memory_space=pltpu.VMEM),
        scratch_shapes=[
            pltpu.VMEM((PB, ZG, m_per, n_per), jnp.float32),
            pltpu.VMEM((ZG, PB * m_per, n_per), jnp.float32),
            pltpu.SemaphoreType.DMA((PB,)),
            pltpu.SemaphoreType.DMA((PB,)),
            pltpu.SemaphoreType.DMA((ZG,)),
            pltpu.SemaphoreType.DMA((ZG,)),
        ],
        compiler_params=pltpu.CompilerParams(collective_id=0),
    )(x, w_mat)


# device time: 25498 ns/iter; 1.4227x vs baseline; 1.4227x over previous
import jax
import jax.numpy as jnp
from jax import lax
from jax.experimental import pallas as pl
from jax.experimental.pallas import tpu as pltpu

N_DEV = 32


def kernel(x, w_mat):
    m_per, k = x.shape
    _, n = w_mat.shape
    n_per = n // N_DEV

    def body(x_ref, w_ref, out_ref, tiles_ref, ssem, rsem):
        my = lax.axis_index("i")

        barrier_sem = pltpu.get_barrier_semaphore()
        for d in range(1, N_DEV):
            pl.semaphore_signal(
                barrier_sem, inc=1,
                device_id=(lax.rem(my + d, N_DEV),),
                device_id_type=pl.DeviceIdType.MESH,
            )

        y = jnp.maximum(
            jnp.dot(x_ref[...], w_ref[...], preferred_element_type=jnp.float32),
            0.0,
        )
        for t in range(N_DEV):
            tiles_ref[t] = y[:, t * n_per:(t + 1) * n_per]

        out_ref[pl.ds(my * m_per, m_per), :] = tiles_ref[my]

        pl.semaphore_wait(barrier_sem, N_DEV - 1)

        for d in range(1, N_DEV):
            tgt = lax.rem(my + d, N_DEV)
            pltpu.make_async_remote_copy(
                src_ref=tiles_ref.at[tgt],
                dst_ref=out_ref.at[pl.ds(my * m_per, m_per), :],
                send_sem=ssem,
                recv_sem=rsem,
                device_id=(tgt,),
                device_id_type=pl.DeviceIdType.MESH,
            ).start()

        agg = out_ref.at[pl.ds(0, (N_DEV - 1) * m_per), :]
        pltpu.make_async_remote_copy(
            src_ref=agg, dst_ref=agg,
            send_sem=ssem, recv_sem=rsem,
            device_id=(my,),
            device_id_type=pl.DeviceIdType.MESH,
        ).wait_recv()

        agg_s = tiles_ref.at[pl.ds(0, N_DEV - 1)]
        pltpu.make_async_remote_copy(
            src_ref=agg_s, dst_ref=agg_s,
            send_sem=ssem, recv_sem=rsem,
            device_id=(my,),
            device_id_type=pl.DeviceIdType.MESH,
        ).wait_send()

    return pl.pallas_call(
        body,
        out_shape=jax.ShapeDtypeStruct((N_DEV * m_per, n_per), jnp.float32),
        in_specs=[
            pl.BlockSpec(memory_space=pltpu.VMEM),
            pl.BlockSpec(memory_space=pltpu.VMEM),
        ],
        out_specs=pl.BlockSpec(memory_space=pltpu.VMEM),
        scratch_shapes=[
            pltpu.VMEM((N_DEV, m_per, n_per), jnp.float32),
            pltpu.SemaphoreType.DMA,
            pltpu.SemaphoreType.DMA,
        ],
        compiler_params=pltpu.CompilerParams(collective_id=0),
    )(x, w_mat)
